# baseline (speedup 1.0000x reference)
import math

import jax
import jax.numpy as jnp
from jax import lax
from jax.experimental import pallas as pl
from jax.experimental.pallas import tpu as pltpu
from jax.experimental.pallas import tpu_sc as plsc

DECAY = 0.9
LOG_D = math.log(DECAY)
B = 16384
M = 100000
L = 16
NW = 32
RANGE = 3136
ROW = 128
NVREG = B // L
MAXC = B + ROW


def _lanes():
    return lax.iota(jnp.int32, L)


def _sc_videosoftmax(
    x_hbm, ids_hbm, mem_hbm, out_hbm,
    idsf, idxf, idx2d, cids, cx, ru, tbl, sem1, sem2,
):
    wid = lax.axis_index("s") * 2 + lax.axis_index("c")
    lo = wid * RANGE
    hi = lo + RANGE
    lanes = _lanes()

    pltpu.sync_copy(ids_hbm, idsf)
    pltpu.sync_copy(mem_hbm.at[pl.ds(0, RANGE)], tbl)

    def scan_step(i, off):
        v = idsf[pl.ds(i * L, L)]
        msk = jnp.logical_and(v >= lo, v < hi)
        pos = lanes + i * L
        plsc.store_compressed(idxf.at[pl.ds(off, L)], pos, mask=msk)
        cnt = plsc.all_reduce_population_count(msk)
        return off + cnt[0]

    n = lax.fori_loop(0, NVREG, scan_step, jnp.int32(0))

    @pl.when(n > 0)
    def _():
        nrows = (n + (ROW - 1)) // ROW
        nv = (n + (L - 1)) // L

        i0 = jnp.full((L,), idxf[pl.ds(0, L)][0], jnp.int32)
        for j in range(ROW // L):
            idxf[pl.ds(n + j * L, L)] = i0

        def cp_row(r, _):
            for j in range(ROW // L):
                idx2d[r, pl.ds(j * L, L)] = idxf[pl.ds(r * ROW + j * L, L)]
            return _

        lax.fori_loop(0, nrows, cp_row, None)

        def gat_row(r, _):
            c1 = pltpu.async_copy(
                ids_hbm.at[idx2d.at[r]], cids.at[pl.ds(r * ROW, ROW)], sem1)
            c2 = pltpu.async_copy(
                x_hbm.at[idx2d.at[r]], cx.at[pl.ds(r * ROW, ROW)], sem2)
            c1.wait()
            c2.wait()
            return _

        lax.fori_loop(0, nrows, gat_row, None)

        def l1(m, _):
            base = m * L
            valid = (lanes + base) < n
            v = cids[pl.ds(base, L)]
            vloc = jnp.clip(v - lo, 0, RANGE - 1)
            veff = jnp.where(valid, v, -1)
            bc = plsc.load_gather(tbl, [vloc])
            p = jnp.zeros((L,), jnp.float32)
            tot = jnp.zeros((L,), jnp.float32)
            for j in range(L):
                eq = veff == jnp.full((L,), veff[j], jnp.int32)
                tot += jnp.where(eq, 1.0, 0.0)
                p += jnp.where(jnp.logical_and(eq, lanes > j), 1.0, 0.0)
            ru[pl.ds(base, L)] = bc + p + 1.0
            plsc.store_scatter(tbl, [vloc], bc + tot, mask=valid)
            return _

        lax.fori_loop(0, nv, l1, None)

        def l2(m, _):
            base = m * L
            v = cids[pl.ds(base, L)]
            vloc = jnp.clip(v - lo, 0, RANGE - 1)
            k = plsc.load_gather(tbl, [vloc])
            r = ru[pl.ds(base, L)]
            c = jnp.minimum(k - r, 300.0)
            f = jnp.where(r <= 1.0, 1.0, 1.0 - DECAY)
            ru[pl.ds(base, L)] = jnp.exp(cx[pl.ds(base, L)] + c * LOG_D) * f
            return _

        lax.fori_loop(0, nv, l2, None)

        pltpu.sync_copy(mem_hbm.at[pl.ds(0, RANGE)], tbl)

        def l3(m, _):
            base = m * L
            valid = (lanes + base) < n
            v = cids[pl.ds(base, L)]
            vloc = jnp.clip(v - lo, 0, RANGE - 1)
            veff = jnp.where(valid, v, -1)
            u = ru[pl.ds(base, L)]
            sb = plsc.load_gather(tbl, [vloc])
            usum = jnp.zeros((L,), jnp.float32)
            for j in range(L):
                eq = veff == jnp.full((L,), veff[j], jnp.int32)
                usum += jnp.where(eq, jnp.full((L,), u[j], jnp.float32), 0.0)
            plsc.store_scatter(tbl, [vloc], sb + usum, mask=valid)
            return _

        lax.fori_loop(0, nv, l3, None)

        def l4(m, _):
            base = m * L
            v = cids[pl.ds(base, L)]
            vloc = jnp.clip(v - lo, 0, RANGE - 1)
            s = plsc.load_gather(tbl, [vloc])
            cx[pl.ds(base, L)] = jnp.exp(cx[pl.ds(base, L)]) / s
            return _

        lax.fori_loop(0, (nrows * ROW) // L, l4, None)

        def sc_row(r, _):
            pltpu.sync_copy(cx.at[pl.ds(r * ROW, ROW)], out_hbm.at[idx2d.at[r]])
            return _

        lax.fori_loop(0, nrows, sc_row, None)


def kernel(input, ids, memory, mask):
    del mask
    mesh = plsc.VectorSubcoreMesh(
        core_axis_name="c", subcore_axis_name="s", num_cores=2, num_subcores=16
    )
    run = pl.kernel(
        _sc_videosoftmax,
        out_type=jax.ShapeDtypeStruct((B,), jnp.float32),
        mesh=mesh,
        compiler_params=pltpu.CompilerParams(needs_layout_passes=False),
        scratch_types=[
            pltpu.VMEM((B,), jnp.int32),
            pltpu.VMEM((MAXC,), jnp.int32),
            pltpu.VMEM((MAXC // ROW, ROW), jnp.int32),
            pltpu.VMEM((MAXC,), jnp.int32),
            pltpu.VMEM((MAXC,), jnp.float32),
            pltpu.VMEM((MAXC,), jnp.float32),
            pltpu.VMEM((RANGE,), jnp.float32),
            pltpu.SemaphoreType.DMA,
            pltpu.SemaphoreType.DMA,
        ],
    )
    return run(input, ids, memory)


# speedup vs baseline: 1.0077x; 1.0077x over previous
import math

import jax
import jax.numpy as jnp
from jax import lax
from jax.experimental import pallas as pl
from jax.experimental.pallas import tpu as pltpu
from jax.experimental.pallas import tpu_sc as plsc

DECAY = 0.9
LOG_D = math.log(DECAY)
B = 16384
M = 100000
L = 16
NT = 16
W = 3136
CHUNK = B // NT
NCV = CHUNK // L
ROW = 128
MAXC = B + ROW


def _lanes():
    return lax.iota(jnp.int32, L)


def _bucket(v):
    return ((v >> 6) * 5350) >> 18


def _sc_videosoftmax(
    x_hbm, ids_hbm, mem_hbm, out_hbm,
    idsch, buck, cnts, cntm, cidx, idx2d, cids, cx, ru, tbl,
    shbuck, shcnt, sem1, sem2,
):
    c = lax.axis_index("c")
    s = lax.axis_index("s")
    lanes = _lanes()

    pltpu.sync_copy(ids_hbm.at[pl.ds(s * CHUNK, CHUNK)], idsch)
    cnts[pl.ds(0, NT)] = jnp.zeros((NT,), jnp.int32)

    def bucketize(i, _):
        v = idsch[pl.ds(i * L, L)]
        gb = _bucket(v)
        mine = (gb >> 4) == c
        lb = jnp.clip(gb & 15, 0, NT - 1)
        lbeff = jnp.where(mine, lb, -1)
        bc = plsc.load_gather(cnts, [lb])
        pfx = jnp.zeros((L,), jnp.int32)
        tot = jnp.zeros((L,), jnp.int32)
        for j in range(L):
            eq = lbeff == jnp.full((L,), lbeff[j], jnp.int32)
            tot += jnp.where(eq, 1, 0)
            pfx += jnp.where(jnp.logical_and(eq, lanes > j), 1, 0)
        o = jnp.clip(bc + pfx, 0, CHUNK - 1)
        pos = lanes + (s * CHUNK + i * L)
        plsc.store_scatter(buck, [lb * CHUNK + o], pos, mask=mine)
        plsc.store_scatter(cnts, [lb], bc + tot, mask=mine)
        return _

    lax.fori_loop(0, NCV, bucketize, None)

    pltpu.sync_copy(buck, shbuck.at[pl.ds(s * (NT * CHUNK), NT * CHUNK)])
    pltpu.sync_copy(cnts, shcnt.at[pl.ds(s * NT, NT)])
    plsc.subcore_barrier()

    base = (c * NT + s) * W
    pltpu.sync_copy(shcnt, cntm)
    cntv = plsc.load_gather(cntm, [lanes * NT + s])
    offv = jnp.zeros((L,), jnp.int32)
    for j in range(L):
        offv += jnp.where(lanes > j, jnp.full((L,), cntv[j], jnp.int32), 0)
    n = offv[L - 1] + cntv[L - 1]

    @pl.when(n > 0)
    def _():
        for p in range(NT):
            pltpu.sync_copy(
                shbuck.at[pl.ds(p * (NT * CHUNK) + s * CHUNK, CHUNK)],
                buck.at[pl.ds(p * CHUNK, CHUNK)])
        for p in range(NT):
            off_p = offv[p]
            nvp = (cntv[p] + (L - 1)) // L

            def cat(j, _, p=p, off_p=off_p):
                cidx[pl.ds(off_p + j * L, L)] = buck[pl.ds(p * CHUNK + j * L, L)]
                return _

            lax.fori_loop(0, nvp, cat, None)

        nrows = (n + (ROW - 1)) // ROW
        nv = (n + (L - 1)) // L

        i0 = jnp.full((L,), cidx[pl.ds(0, L)][0], jnp.int32)
        for j in range(ROW // L):
            cidx[pl.ds(n + j * L, L)] = i0

        def cp_row(r, _):
            for j in range(ROW // L):
                idx2d[r, pl.ds(j * L, L)] = cidx[pl.ds(r * ROW + j * L, L)]
            return _

        lax.fori_loop(0, nrows, cp_row, None)

        def gat_row(r, _):
            c1 = pltpu.async_copy(
                ids_hbm.at[idx2d.at[r]], cids.at[pl.ds(r * ROW, ROW)], sem1)
            c2 = pltpu.async_copy(
                x_hbm.at[idx2d.at[r]], cx.at[pl.ds(r * ROW, ROW)], sem2)
            c1.wait()
            c2.wait()
            return _

        lax.fori_loop(0, nrows, gat_row, None)

        pltpu.sync_copy(mem_hbm.at[pl.ds(0, W)], tbl)

        def l1(m, _):
            b0 = m * L
            valid = (lanes + b0) < n
            v = cids[pl.ds(b0, L)]
            vloc = jnp.clip(v - base, 0, W - 1)
            veff = jnp.where(valid, v, -1)
            bc = plsc.load_gather(tbl, [vloc])
            p = jnp.zeros((L,), jnp.float32)
            tot = jnp.zeros((L,), jnp.float32)
            for j in range(L):
                eq = veff == jnp.full((L,), veff[j], jnp.int32)
                tot += jnp.where(eq, 1.0, 0.0)
                p += jnp.where(jnp.logical_and(eq, lanes > j), 1.0, 0.0)
            ru[pl.ds(b0, L)] = bc + p + 1.0
            plsc.store_scatter(tbl, [vloc], bc + tot, mask=valid)
            return _

        lax.fori_loop(0, nv, l1, None)

        def l2(m, _):
            b0 = m * L
            v = cids[pl.ds(b0, L)]
            vloc = jnp.clip(v - base, 0, W - 1)
            k = plsc.load_gather(tbl, [vloc])
            r = ru[pl.ds(b0, L)]
            cc = jnp.minimum(k - r, 300.0)
            f = jnp.where(r <= 1.0, 1.0, 1.0 - DECAY)
            ru[pl.ds(b0, L)] = jnp.exp(cx[pl.ds(b0, L)] + cc * LOG_D) * f
            return _

        lax.fori_loop(0, nv, l2, None)

        pltpu.sync_copy(mem_hbm.at[pl.ds(0, W)], tbl)

        def l3(m, _):
            b0 = m * L
            valid = (lanes + b0) < n
            v = cids[pl.ds(b0, L)]
            vloc = jnp.clip(v - base, 0, W - 1)
            veff = jnp.where(valid, v, -1)
            u = ru[pl.ds(b0, L)]
            sb = plsc.load_gather(tbl, [vloc])
            usum = jnp.zeros((L,), jnp.float32)
            for j in range(L):
                eq = veff == jnp.full((L,), veff[j], jnp.int32)
                usum += jnp.where(eq, jnp.full((L,), u[j], jnp.float32), 0.0)
            plsc.store_scatter(tbl, [vloc], sb + usum, mask=valid)
            return _

        lax.fori_loop(0, nv, l3, None)

        def l4(m, _):
            b0 = m * L
            v = cids[pl.ds(b0, L)]
            vloc = jnp.clip(v - base, 0, W - 1)
            sv = plsc.load_gather(tbl, [vloc])
            cx[pl.ds(b0, L)] = jnp.exp(cx[pl.ds(b0, L)]) / sv
            return _

        lax.fori_loop(0, (nrows * ROW) // L, l4, None)

        def sc_row(r, _):
            pltpu.sync_copy(cx.at[pl.ds(r * ROW, ROW)], out_hbm.at[idx2d.at[r]])
            return _

        lax.fori_loop(0, nrows, sc_row, None)


def kernel(input, ids, memory, mask):
    del mask
    mesh = plsc.VectorSubcoreMesh(
        core_axis_name="c", subcore_axis_name="s", num_cores=2, num_subcores=16
    )
    run = pl.kernel(
        _sc_videosoftmax,
        out_type=jax.ShapeDtypeStruct((B,), jnp.float32),
        mesh=mesh,
        compiler_params=pltpu.CompilerParams(needs_layout_passes=False),
        scratch_types=[
            pltpu.VMEM((CHUNK,), jnp.int32),
            pltpu.VMEM((NT * CHUNK,), jnp.int32),
            pltpu.VMEM((NT,), jnp.int32),
            pltpu.VMEM((NT * NT,), jnp.int32),
            pltpu.VMEM((MAXC,), jnp.int32),
            pltpu.VMEM((MAXC // ROW, ROW), jnp.int32),
            pltpu.VMEM((MAXC,), jnp.int32),
            pltpu.VMEM((MAXC,), jnp.float32),
            pltpu.VMEM((MAXC,), jnp.float32),
            pltpu.VMEM((W,), jnp.float32),
            pltpu.VMEM_SHARED((NT * NT * CHUNK,), jnp.int32),
            pltpu.VMEM_SHARED((NT * NT,), jnp.int32),
            pltpu.SemaphoreType.DMA,
            pltpu.SemaphoreType.DMA,
        ],
    )
    return run(input, ids, memory)


# speedup vs baseline: 1.0087x; 1.0011x over previous
import math

import jax
import jax.numpy as jnp
from jax import lax
from jax.experimental import pallas as pl
from jax.experimental.pallas import tpu as pltpu
from jax.experimental.pallas import tpu_sc as plsc

DECAY = 0.9
LOG_D = math.log(DECAY)
B = 16384
M = 100000
L = 16
NT = 16
W = 3136
CHUNK = B // NT
NCV = CHUNK // L
ROW = 128
MAXC = B + ROW


def _lanes():
    return lax.iota(jnp.int32, L)


def _bucket(v):
    return ((v >> 6) * 5350) >> 18


def _sc_videosoftmax(
    x_hbm, ids_hbm, mem_hbm, out_hbm,
    idsch, buck, cnts, cntm, cidx, idx2d, cids, cx, ru, tbl,
    shbuck, shcnt, sem1, sem2,
):
    c = lax.axis_index("c")
    s = lax.axis_index("s")
    lanes = _lanes()

    with jax.named_scope("ph_a_stage"):
        pltpu.sync_copy(ids_hbm.at[pl.ds(s * CHUNK, CHUNK)], idsch)
    cnts[pl.ds(0, NT)] = jnp.zeros((NT,), jnp.int32)

    def bucketize(i, _):
        v = idsch[pl.ds(i * L, L)]
        gb = _bucket(v)
        mine = (gb >> 4) == c
        lb = jnp.clip(gb & 15, 0, NT - 1)
        lbeff = jnp.where(mine, lb, -1)
        bc = plsc.load_gather(cnts, [lb])
        pfx = jnp.zeros((L,), jnp.int32)
        tot = jnp.zeros((L,), jnp.int32)
        for j in range(L):
            eq = lbeff == jnp.full((L,), lbeff[j], jnp.int32)
            tot += jnp.where(eq, 1, 0)
            pfx += jnp.where(jnp.logical_and(eq, lanes > j), 1, 0)
        o = jnp.clip(bc + pfx, 0, CHUNK - 1)
        pos = lanes + (s * CHUNK + i * L)
        plsc.store_scatter(buck, [lb * CHUNK + o], pos, mask=mine)
        plsc.store_scatter(cnts, [lb], bc + tot, mask=mine)
        return _

    with jax.named_scope("ph_a_scan"):
        lax.fori_loop(0, NCV, bucketize, None)

    with jax.named_scope("ph_publish"):
        pltpu.sync_copy(buck, shbuck.at[pl.ds(s * (NT * CHUNK), NT * CHUNK)])
        pltpu.sync_copy(cnts, shcnt.at[pl.ds(s * NT, NT)])
    with jax.named_scope("ph_barrier"):
        plsc.subcore_barrier()

    base = (c * NT + s) * W
    pltpu.sync_copy(shcnt, cntm)
    cntv = plsc.load_gather(cntm, [lanes * NT + s])
    offv = jnp.zeros((L,), jnp.int32)
    for j in range(L):
        offv += jnp.where(lanes > j, jnp.full((L,), cntv[j], jnp.int32), 0)
    n = offv[L - 1] + cntv[L - 1]

    @pl.when(n > 0)
    def _():
        with jax.named_scope("ph_consume_copy"):
            for p in range(NT):
                pltpu.sync_copy(
                    shbuck.at[pl.ds(p * (NT * CHUNK) + s * CHUNK, CHUNK)],
                    buck.at[pl.ds(p * CHUNK, CHUNK)])
        with jax.named_scope("ph_concat"):
            for p in range(NT):
                off_p = offv[p]
                nvp = (cntv[p] + (L - 1)) // L

                def cat(j, _, p=p, off_p=off_p):
                    cidx[pl.ds(off_p + j * L, L)] = buck[
                        pl.ds(p * CHUNK + j * L, L)]
                    return _

                lax.fori_loop(0, nvp, cat, None)

        nrows = (n + (ROW - 1)) // ROW
        nv = (n + (L - 1)) // L

        i0 = jnp.full((L,), cidx[pl.ds(0, L)][0], jnp.int32)
        for j in range(ROW // L):
            cidx[pl.ds(n + j * L, L)] = i0

        def cp_row(r, _):
            for j in range(ROW // L):
                idx2d[r, pl.ds(j * L, L)] = cidx[pl.ds(r * ROW + j * L, L)]
            return _

        with jax.named_scope("ph_rows"):
            lax.fori_loop(0, nrows, cp_row, None)

        def gat_row(r, _):
            c1 = pltpu.async_copy(
                ids_hbm.at[idx2d.at[r]], cids.at[pl.ds(r * ROW, ROW)], sem1)
            c2 = pltpu.async_copy(
                x_hbm.at[idx2d.at[r]], cx.at[pl.ds(r * ROW, ROW)], sem2)
            c1.wait()
            c2.wait()
            return _

        with jax.named_scope("ph_gather"):
            lax.fori_loop(0, nrows, gat_row, None)

        pltpu.sync_copy(mem_hbm.at[pl.ds(0, W)], tbl)

        def l1(m, _):
            b0 = m * L
            valid = (lanes + b0) < n
            v = cids[pl.ds(b0, L)]
            vloc = jnp.clip(v - base, 0, W - 1)
            veff = jnp.where(valid, v, -1)
            bc = plsc.load_gather(tbl, [vloc])
            p = jnp.zeros((L,), jnp.float32)
            tot = jnp.zeros((L,), jnp.float32)
            for j in range(L):
                eq = veff == jnp.full((L,), veff[j], jnp.int32)
                tot += jnp.where(eq, 1.0, 0.0)
                p += jnp.where(jnp.logical_and(eq, lanes > j), 1.0, 0.0)
            ru[pl.ds(b0, L)] = bc + p + 1.0
            plsc.store_scatter(tbl, [vloc], bc + tot, mask=valid)
            return _

        with jax.named_scope("ph_l1"):
            lax.fori_loop(0, nv, l1, None)

        def l2(m, _):
            b0 = m * L
            v = cids[pl.ds(b0, L)]
            vloc = jnp.clip(v - base, 0, W - 1)
            k = plsc.load_gather(tbl, [vloc])
            r = ru[pl.ds(b0, L)]
            cc = jnp.minimum(k - r, 300.0)
            f = jnp.where(r <= 1.0, 1.0, 1.0 - DECAY)
            ru[pl.ds(b0, L)] = jnp.exp(cx[pl.ds(b0, L)] + cc * LOG_D) * f
            return _

        with jax.named_scope("ph_l2"):
            lax.fori_loop(0, nv, l2, None)

        pltpu.sync_copy(mem_hbm.at[pl.ds(0, W)], tbl)

        def l3(m, _):
            b0 = m * L
            valid = (lanes + b0) < n
            v = cids[pl.ds(b0, L)]
            vloc = jnp.clip(v - base, 0, W - 1)
            veff = jnp.where(valid, v, -1)
            u = ru[pl.ds(b0, L)]
            sb = plsc.load_gather(tbl, [vloc])
            usum = jnp.zeros((L,), jnp.float32)
            for j in range(L):
                eq = veff == jnp.full((L,), veff[j], jnp.int32)
                usum += jnp.where(eq, jnp.full((L,), u[j], jnp.float32), 0.0)
            plsc.store_scatter(tbl, [vloc], sb + usum, mask=valid)
            return _

        with jax.named_scope("ph_l3"):
            lax.fori_loop(0, nv, l3, None)

        def l4(m, _):
            b0 = m * L
            v = cids[pl.ds(b0, L)]
            vloc = jnp.clip(v - base, 0, W - 1)
            sv = plsc.load_gather(tbl, [vloc])
            cx[pl.ds(b0, L)] = jnp.exp(cx[pl.ds(b0, L)]) / sv
            return _

        with jax.named_scope("ph_l4"):
            lax.fori_loop(0, (nrows * ROW) // L, l4, None)

        def sc_row(r, _):
            pltpu.sync_copy(cx.at[pl.ds(r * ROW, ROW)], out_hbm.at[idx2d.at[r]])
            return _

        with jax.named_scope("ph_scatter"):
            lax.fori_loop(0, nrows, sc_row, None)


def kernel(input, ids, memory, mask):
    del mask
    mesh = plsc.VectorSubcoreMesh(
        core_axis_name="c", subcore_axis_name="s", num_cores=2, num_subcores=16
    )
    run = pl.kernel(
        _sc_videosoftmax,
        out_type=jax.ShapeDtypeStruct((B,), jnp.float32),
        mesh=mesh,
        compiler_params=pltpu.CompilerParams(needs_layout_passes=False),
        scratch_types=[
            pltpu.VMEM((CHUNK,), jnp.int32),
            pltpu.VMEM((NT * CHUNK,), jnp.int32),
            pltpu.VMEM((NT,), jnp.int32),
            pltpu.VMEM((NT * NT,), jnp.int32),
            pltpu.VMEM((MAXC,), jnp.int32),
            pltpu.VMEM((MAXC // ROW, ROW), jnp.int32),
            pltpu.VMEM((MAXC,), jnp.int32),
            pltpu.VMEM((MAXC,), jnp.float32),
            pltpu.VMEM((MAXC,), jnp.float32),
            pltpu.VMEM((W,), jnp.float32),
            pltpu.VMEM_SHARED((NT * NT * CHUNK,), jnp.int32),
            pltpu.VMEM_SHARED((NT * NT,), jnp.int32),
            pltpu.SemaphoreType.DMA,
            pltpu.SemaphoreType.DMA,
        ],
    )
    return run(input, ids, memory)


# speedup vs baseline: 8.1122x; 8.0420x over previous
import math

import jax
import jax.numpy as jnp
from jax import lax
from jax.experimental import pallas as pl
from jax.experimental.pallas import tpu as pltpu
from jax.experimental.pallas import tpu_sc as plsc

DECAY = 0.9
LOG_D = math.log(DECAY)
B = 16384
M = 100000
L = 16
NT = 16
W = 3136
MS = 32 * W
CHUNK = B // NT
NCV = CHUNK // L
CH2 = B // 32
MAXC = B + L


def _lanes():
    return lax.iota(jnp.int32, L)


def _bucket(v):
    return ((v >> 6) * 5350) >> 18


def _sc_segsum(
    x_hbm, ids_hbm, mem_hbm, s_hbm,
    idsch, xch, buckid, buckx, cnts, cntm, cids, cx, ru, tbl,
    shid, shx, shcnt,
):
    c = lax.axis_index("c")
    s = lax.axis_index("s")
    lanes = _lanes()

    pltpu.sync_copy(ids_hbm.at[pl.ds(s * CHUNK, CHUNK)], idsch)
    pltpu.sync_copy(x_hbm.at[pl.ds(s * CHUNK, CHUNK)], xch)
    cnts[pl.ds(0, NT)] = jnp.zeros((NT,), jnp.int32)

    def bucketize(i, _):
        v = idsch[pl.ds(i * L, L)]
        xv = xch[pl.ds(i * L, L)]
        gb = _bucket(v)
        mine = (gb >> 4) == c
        lb = jnp.clip(gb & 15, 0, NT - 1)
        lbeff = jnp.where(mine, lb, -1)
        bc = plsc.load_gather(cnts, [lb])
        pfx = jnp.zeros((L,), jnp.int32)
        tot = jnp.zeros((L,), jnp.int32)
        for j in range(L):
            eq = lbeff == jnp.full((L,), lbeff[j], jnp.int32)
            tot += jnp.where(eq, 1, 0)
            pfx += jnp.where(jnp.logical_and(eq, lanes > j), 1, 0)
        o = lb * CHUNK + jnp.clip(bc + pfx, 0, CHUNK - 1)
        plsc.store_scatter(buckid, [o], v, mask=mine)
        plsc.store_scatter(buckx, [o], xv, mask=mine)
        plsc.store_scatter(cnts, [lb], bc + tot, mask=mine)
        return _

    lax.fori_loop(0, NCV, bucketize, None)

    pltpu.sync_copy(buckid, shid.at[pl.ds(s * (NT * CHUNK), NT * CHUNK)])
    pltpu.sync_copy(buckx, shx.at[pl.ds(s * (NT * CHUNK), NT * CHUNK)])
    pltpu.sync_copy(cnts, shcnt.at[pl.ds(s * NT, NT)])
    plsc.subcore_barrier()

    base = (c * NT + s) * W
    pltpu.sync_copy(shcnt, cntm)
    pltpu.sync_copy(mem_hbm.at[pl.ds(0, W)], tbl)
    cntv = plsc.load_gather(cntm, [lanes * NT + s])
    offv = jnp.zeros((L,), jnp.int32)
    for j in range(L):
        offv += jnp.where(lanes > j, jnp.full((L,), cntv[j], jnp.int32), 0)
    n = offv[L - 1] + cntv[L - 1]

    @pl.when(n > 0)
    def _():
        for p in range(NT):
            pltpu.sync_copy(
                shid.at[pl.ds(p * (NT * CHUNK) + s * CHUNK, CHUNK)],
                buckid.at[pl.ds(p * CHUNK, CHUNK)])
            pltpu.sync_copy(
                shx.at[pl.ds(p * (NT * CHUNK) + s * CHUNK, CHUNK)],
                buckx.at[pl.ds(p * CHUNK, CHUNK)])
        for p in range(NT):
            off_p = offv[p]
            nvp = (cntv[p] + (L - 1)) // L

            def cat(j, _, p=p, off_p=off_p):
                cids[pl.ds(off_p + j * L, L)] = buckid[
                    pl.ds(p * CHUNK + j * L, L)]
                cx[pl.ds(off_p + j * L, L)] = buckx[
                    pl.ds(p * CHUNK + j * L, L)]
                return _

            lax.fori_loop(0, nvp, cat, None)

        nv = (n + (L - 1)) // L

        def l1(m, _):
            b0 = m * L
            valid = (lanes + b0) < n
            v = cids[pl.ds(b0, L)]
            vloc = jnp.clip(v - base, 0, W - 1)
            veff = jnp.where(valid, v, -1)
            bc = plsc.load_gather(tbl, [vloc])
            p = jnp.zeros((L,), jnp.float32)
            tot = jnp.zeros((L,), jnp.float32)
            for j in range(L):
                eq = veff == jnp.full((L,), veff[j], jnp.int32)
                tot += jnp.where(eq, 1.0, 0.0)
                p += jnp.where(jnp.logical_and(eq, lanes > j), 1.0, 0.0)
            ru[pl.ds(b0, L)] = bc + p + 1.0
            plsc.store_scatter(tbl, [vloc], bc + tot, mask=valid)
            return _

        lax.fori_loop(0, nv, l1, None)

        def l2(m, _):
            b0 = m * L
            v = cids[pl.ds(b0, L)]
            vloc = jnp.clip(v - base, 0, W - 1)
            k = plsc.load_gather(tbl, [vloc])
            r = ru[pl.ds(b0, L)]
            cc = jnp.minimum(k - r, 300.0)
            f = jnp.where(r <= 1.0, 1.0, 1.0 - DECAY)
            ru[pl.ds(b0, L)] = jnp.exp(cx[pl.ds(b0, L)] + cc * LOG_D) * f
            return _

        lax.fori_loop(0, nv, l2, None)

        pltpu.sync_copy(mem_hbm.at[pl.ds(0, W)], tbl)

        def l3(m, _):
            b0 = m * L
            valid = (lanes + b0) < n
            v = cids[pl.ds(b0, L)]
            vloc = jnp.clip(v - base, 0, W - 1)
            veff = jnp.where(valid, v, -1)
            u = ru[pl.ds(b0, L)]
            sb = plsc.load_gather(tbl, [vloc])
            usum = jnp.zeros((L,), jnp.float32)
            for j in range(L):
                eq = veff == jnp.full((L,), veff[j], jnp.int32)
                usum += jnp.where(eq, jnp.full((L,), u[j], jnp.float32), 0.0)
            plsc.store_scatter(tbl, [vloc], sb + usum, mask=valid)
            return _

        lax.fori_loop(0, nv, l3, None)

    pltpu.sync_copy(tbl, s_hbm.at[pl.ds(base, W)])


def _sc_apply(x_hbm, ids_hbm, s_hbm, out_hbm, stbl, idsch, xch, och):
    c = lax.axis_index("c")
    s = lax.axis_index("s")
    base = (s * 2 + c) * CH2

    pltpu.sync_copy(s_hbm, stbl)
    pltpu.sync_copy(ids_hbm.at[pl.ds(base, CH2)], idsch)
    pltpu.sync_copy(x_hbm.at[pl.ds(base, CH2)], xch)

    def apply_step(m, _):
        b0 = m * L
        v = idsch[pl.ds(b0, L)]
        sv = plsc.load_gather(stbl, [v])
        och[pl.ds(b0, L)] = jnp.exp(xch[pl.ds(b0, L)]) / sv
        return _

    lax.fori_loop(0, CH2 // L, apply_step, None)
    pltpu.sync_copy(och, out_hbm.at[pl.ds(base, CH2)])


def kernel(input, ids, memory, mask):
    del mask
    mesh = plsc.VectorSubcoreMesh(
        core_axis_name="c", subcore_axis_name="s", num_cores=2, num_subcores=16
    )
    segsum = pl.kernel(
        _sc_segsum,
        out_type=jax.ShapeDtypeStruct((MS,), jnp.float32),
        mesh=mesh,
        compiler_params=pltpu.CompilerParams(needs_layout_passes=False),
        scratch_types=[
            pltpu.VMEM((CHUNK,), jnp.int32),
            pltpu.VMEM((CHUNK,), jnp.float32),
            pltpu.VMEM((NT * CHUNK,), jnp.int32),
            pltpu.VMEM((NT * CHUNK,), jnp.float32),
            pltpu.VMEM((NT,), jnp.int32),
            pltpu.VMEM((NT * NT,), jnp.int32),
            pltpu.VMEM((MAXC,), jnp.int32),
            pltpu.VMEM((MAXC,), jnp.float32),
            pltpu.VMEM((MAXC,), jnp.float32),
            pltpu.VMEM((W,), jnp.float32),
            pltpu.VMEM_SHARED((NT * NT * CHUNK,), jnp.int32),
            pltpu.VMEM_SHARED((NT * NT * CHUNK,), jnp.float32),
            pltpu.VMEM_SHARED((NT * NT,), jnp.int32),
        ],
    )
    apply = pl.kernel(
        _sc_apply,
        out_type=jax.ShapeDtypeStruct((B,), jnp.float32),
        mesh=mesh,
        compiler_params=pltpu.CompilerParams(needs_layout_passes=False),
        scratch_types=[
            pltpu.VMEM((MS,), jnp.float32),
            pltpu.VMEM((CH2,), jnp.int32),
            pltpu.VMEM((CH2,), jnp.float32),
            pltpu.VMEM((CH2,), jnp.float32),
        ],
    )
    s_all = segsum(input, ids, memory)
    return apply(input, ids, s_all)


# speedup vs baseline: 8.7657x; 1.0806x over previous
import math

import jax
import jax.numpy as jnp
from jax import lax
from jax.experimental import pallas as pl
from jax.experimental.pallas import tpu as pltpu
from jax.experimental.pallas import tpu_sc as plsc

DECAY = 0.9
LOG_D = math.log(DECAY)
B = 16384
M = 100000
L = 16
NT = 16
W = 3136
MS = 32 * W
CHUNK = B // NT
NCV = CHUNK // L
CH2 = B // 32
MAXC = B + L


def _lanes():
    return lax.iota(jnp.int32, L)


def _bucket(v):
    return ((v >> 6) * 5350) >> 18


def _sc_segsum(
    x_hbm, ids_hbm, mem_hbm, s_hbm,
    idsch, xch, buckid, buckx, cnts, cntm, cids, cx, ru, tbl,
    shid, shx, shcnt, sem1, sem2,
):
    c = lax.axis_index("c")
    s = lax.axis_index("s")
    lanes = _lanes()

    pltpu.sync_copy(ids_hbm.at[pl.ds(s * CHUNK, CHUNK)], idsch)
    pltpu.sync_copy(x_hbm.at[pl.ds(s * CHUNK, CHUNK)], xch)
    cnts[pl.ds(0, NT)] = jnp.zeros((NT,), jnp.int32)

    def bucketize(i, _):
        v = idsch[pl.ds(i * L, L)]
        xv = xch[pl.ds(i * L, L)]
        gb = _bucket(v)
        mine = (gb >> 4) == c
        lb = jnp.clip(gb & 15, 0, NT - 1)
        lbeff = jnp.where(mine, lb, -1)
        bc = plsc.load_gather(cnts, [lb])
        pfx = jnp.zeros((L,), jnp.int32)
        tot = jnp.zeros((L,), jnp.int32)
        for j in range(L):
            eq = lbeff == jnp.full((L,), lbeff[j], jnp.int32)
            tot += jnp.where(eq, 1, 0)
            pfx += jnp.where(jnp.logical_and(eq, lanes > j), 1, 0)
        o = lb * CHUNK + jnp.clip(bc + pfx, 0, CHUNK - 1)
        plsc.store_scatter(buckid, [o], v, mask=mine)
        plsc.store_scatter(buckx, [o], xv, mask=mine)
        plsc.store_scatter(cnts, [lb], bc + tot, mask=mine)
        return _

    lax.fori_loop(0, NCV, bucketize, None)

    p1 = pltpu.async_copy(
        buckid, shid.at[pl.ds(s * (NT * CHUNK), NT * CHUNK)], sem1)
    p2 = pltpu.async_copy(
        buckx, shx.at[pl.ds(s * (NT * CHUNK), NT * CHUNK)], sem2)
    pltpu.sync_copy(cnts, shcnt.at[pl.ds(s * NT, NT)])
    p1.wait()
    p2.wait()
    plsc.subcore_barrier()

    base = (c * NT + s) * W
    pltpu.sync_copy(shcnt, cntm)
    pltpu.sync_copy(mem_hbm.at[pl.ds(0, W)], tbl)
    cntv = plsc.load_gather(cntm, [lanes * NT + s])
    offv = jnp.zeros((L,), jnp.int32)
    for j in range(L):
        offv += jnp.where(lanes > j, jnp.full((L,), cntv[j], jnp.int32), 0)
    n = offv[L - 1] + cntv[L - 1]

    @pl.when(n > 0)
    def _():
        cps = []
        for p in range(NT):
            cps.append(pltpu.async_copy(
                shid.at[pl.ds(p * (NT * CHUNK) + s * CHUNK, CHUNK)],
                buckid.at[pl.ds(p * CHUNK, CHUNK)], sem1))
            cps.append(pltpu.async_copy(
                shx.at[pl.ds(p * (NT * CHUNK) + s * CHUNK, CHUNK)],
                buckx.at[pl.ds(p * CHUNK, CHUNK)], sem2))
        for cp in cps:
            cp.wait()
        for p in range(NT):
            off_p = offv[p]
            nvp = (cntv[p] + (L - 1)) // L

            def cat(j, _, p=p, off_p=off_p):
                cids[pl.ds(off_p + j * L, L)] = buckid[
                    pl.ds(p * CHUNK + j * L, L)]
                cx[pl.ds(off_p + j * L, L)] = buckx[
                    pl.ds(p * CHUNK + j * L, L)]
                return _

            lax.fori_loop(0, nvp, cat, None)

        nv = (n + (L - 1)) // L

        def l1(m, _):
            b0 = m * L
            valid = (lanes + b0) < n
            v = cids[pl.ds(b0, L)]
            vloc = jnp.clip(v - base, 0, W - 1)
            veff = jnp.where(valid, v, -1)
            bc = plsc.load_gather(tbl, [vloc])
            p = jnp.zeros((L,), jnp.float32)
            tot = jnp.zeros((L,), jnp.float32)
            for j in range(L):
                eq = veff == jnp.full((L,), veff[j], jnp.int32)
                tot += jnp.where(eq, 1.0, 0.0)
                p += jnp.where(jnp.logical_and(eq, lanes > j), 1.0, 0.0)
            ru[pl.ds(b0, L)] = bc + p + 1.0
            plsc.store_scatter(tbl, [vloc], bc + tot, mask=valid)
            return _

        lax.fori_loop(0, nv, l1, None)

        def l2(m, _):
            b0 = m * L
            v = cids[pl.ds(b0, L)]
            vloc = jnp.clip(v - base, 0, W - 1)
            k = plsc.load_gather(tbl, [vloc])
            r = ru[pl.ds(b0, L)]
            cc = jnp.minimum(k - r, 300.0)
            f = jnp.where(r <= 1.0, 1.0, 1.0 - DECAY)
            ru[pl.ds(b0, L)] = jnp.exp(cx[pl.ds(b0, L)] + cc * LOG_D) * f
            return _

        lax.fori_loop(0, nv, l2, None)

        pltpu.sync_copy(mem_hbm.at[pl.ds(0, W)], tbl)

        def l3(m, _):
            b0 = m * L
            valid = (lanes + b0) < n
            v = cids[pl.ds(b0, L)]
            vloc = jnp.clip(v - base, 0, W - 1)
            veff = jnp.where(valid, v, -1)
            u = ru[pl.ds(b0, L)]
            sb = plsc.load_gather(tbl, [vloc])
            usum = jnp.zeros((L,), jnp.float32)
            for j in range(L):
                eq = veff == jnp.full((L,), veff[j], jnp.int32)
                usum += jnp.where(eq, jnp.full((L,), u[j], jnp.float32), 0.0)
            plsc.store_scatter(tbl, [vloc], sb + usum, mask=valid)
            return _

        lax.fori_loop(0, nv, l3, None)

    pltpu.sync_copy(tbl, s_hbm.at[pl.ds(base, W)])


def _sc_apply(x_hbm, ids_hbm, s_hbm, out_hbm, stbl, idsch, xch, och, sem1,
              sem2):
    c = lax.axis_index("c")
    s = lax.axis_index("s")
    base = (s * 2 + c) * CH2

    c1 = pltpu.async_copy(s_hbm, stbl, sem1)
    c2 = pltpu.async_copy(ids_hbm.at[pl.ds(base, CH2)], idsch, sem2)
    pltpu.sync_copy(x_hbm.at[pl.ds(base, CH2)], xch)
    c1.wait()
    c2.wait()

    def apply_step(m, _):
        b0 = m * L
        v = idsch[pl.ds(b0, L)]
        sv = plsc.load_gather(stbl, [v])
        och[pl.ds(b0, L)] = jnp.exp(xch[pl.ds(b0, L)]) / sv
        return _

    lax.fori_loop(0, CH2 // L, apply_step, None)
    pltpu.sync_copy(och, out_hbm.at[pl.ds(base, CH2)])


def kernel(input, ids, memory, mask):
    del mask
    mesh = plsc.VectorSubcoreMesh(
        core_axis_name="c", subcore_axis_name="s", num_cores=2, num_subcores=16
    )
    segsum = pl.kernel(
        _sc_segsum,
        out_type=jax.ShapeDtypeStruct((MS,), jnp.float32),
        mesh=mesh,
        compiler_params=pltpu.CompilerParams(needs_layout_passes=False),
        scratch_types=[
            pltpu.VMEM((CHUNK,), jnp.int32),
            pltpu.VMEM((CHUNK,), jnp.float32),
            pltpu.VMEM((NT * CHUNK,), jnp.int32),
            pltpu.VMEM((NT * CHUNK,), jnp.float32),
            pltpu.VMEM((NT,), jnp.int32),
            pltpu.VMEM((NT * NT,), jnp.int32),
            pltpu.VMEM((MAXC,), jnp.int32),
            pltpu.VMEM((MAXC,), jnp.float32),
            pltpu.VMEM((MAXC,), jnp.float32),
            pltpu.VMEM((W,), jnp.float32),
            pltpu.VMEM_SHARED((NT * NT * CHUNK,), jnp.int32),
            pltpu.VMEM_SHARED((NT * NT * CHUNK,), jnp.float32),
            pltpu.VMEM_SHARED((NT * NT,), jnp.int32),
            pltpu.SemaphoreType.DMA,
            pltpu.SemaphoreType.DMA,
        ],
    )
    apply = pl.kernel(
        _sc_apply,
        out_type=jax.ShapeDtypeStruct((B,), jnp.float32),
        mesh=mesh,
        compiler_params=pltpu.CompilerParams(needs_layout_passes=False),
        scratch_types=[
            pltpu.VMEM((MS,), jnp.float32),
            pltpu.VMEM((CH2,), jnp.int32),
            pltpu.VMEM((CH2,), jnp.float32),
            pltpu.VMEM((CH2,), jnp.float32),
            pltpu.SemaphoreType.DMA,
            pltpu.SemaphoreType.DMA,
        ],
    )
    s_all = segsum(input, ids, memory)
    return apply(input, ids, s_all)
